# Initial kernel scaffold; baseline (speedup 1.0000x reference)
import jax
import jax.numpy as jnp
from jax.experimental import pallas as pl


def kernel(edge_features, edge_index, W0, b0, W1, b1, W2, b2, We, be):
    raise NotImplementedError("write your pallas kernel here")


# speedup vs baseline: 3.6274x; 3.6274x over previous
import jax
import jax.numpy as jnp
from jax import lax
from jax.experimental import pallas as pl
from jax.experimental.pallas import tpu as pltpu
from jax.experimental.pallas import tpu_sc as plsc

NN = 10000
NNP = 10240
NE = 320000
D = 128
NC = 2
NS = 16
EPT = NE // (NC * NS)
EC = 80
NCHUNK = EPT // EC
RPT = NNP // NS
CW = 128
VL = 16

_mesh = plsc.VectorSubcoreMesh(
    core_axis_name="c", subcore_axis_name="s", num_cores=NC, num_subcores=NS
)


def _zero_rows(buf, nrows, ncols):
    z = jnp.zeros((VL,), jnp.float32)
    nvec = ncols // VL

    def body(i, _):
        buf[i // nvec, pl.ds((i % nvec) * VL, VL)] = z
        return 0

    lax.fori_loop(0, nrows * nvec, body, 0)


def _zero_acc(acc, sub, zbuf):
    for k in range(RPT // 128):
        pltpu.sync_copy(zbuf, acc.at[pl.ds(sub * RPT + k * 128, 128)])


def _copy_acc_out(acc, out, core, sub, zbuf):
    for k in range(RPT // 128):
        r0 = sub * RPT + k * 128
        pltpu.sync_copy(acc.at[pl.ds(r0, 128)], zbuf)
        pltpu.sync_copy(zbuf, out.at[pl.ds(core * NNP + r0, 128)])


def _sc_count(dstf, cnts, ones, idxr, zcnt, cac):
    core = lax.axis_index("c")
    sub = lax.axis_index("s")
    tile_off = (core * NS + sub) * EPT

    _zero_rows(zcnt, 128, CW)
    for k in range(RPT // 128):
        pltpu.sync_copy(zcnt, cac.at[pl.ds(sub * RPT + k * 128, 128)])
    one = jnp.ones((VL,), jnp.float32)

    nv = CW // VL

    def ones_body(i, _):
        ones[i // nv, pl.ds((i % nv) * VL, VL)] = one
        return 0

    lax.fori_loop(0, EC * nv, ones_body, 0)
    plsc.subcore_barrier()

    def chunk(c, _):
        base = tile_off + c * EC
        pltpu.sync_copy(dstf.at[pl.ds(base, EC)], idxr)
        pltpu.sync_copy(ones, cac.at[idxr], add=True)
        return 0

    lax.fori_loop(0, NCHUNK, chunk, 0)
    plsc.subcore_barrier()
    for k in range(RPT // 128):
        r0 = sub * RPT + k * 128
        pltpu.sync_copy(cac.at[pl.ds(r0, 128)], zcnt)
        pltpu.sync_copy(zcnt, cnts.at[pl.ds(core * NNP + r0, 128)])


def _sc_init(ef, dstf, sums, rows, idxr, zbuf, acc, sem):
    core = lax.axis_index("c")
    sub = lax.axis_index("s")
    tile_off = (core * NS + sub) * EPT

    _zero_rows(zbuf, 128, D)
    _zero_acc(acc, sub, zbuf)
    plsc.subcore_barrier()

    def chunk(c, _):
        base = tile_off + c * EC
        pltpu.sync_copy(dstf.at[pl.ds(base, EC)], idxr)
        pltpu.async_copy(ef.at[pl.ds(base, EC)], rows, sem).wait()
        pltpu.sync_copy(rows, acc.at[idxr], add=True)
        return 0

    lax.fori_loop(0, NCHUNK, chunk, 0)
    plsc.subcore_barrier()
    _copy_acc_out(acc, sums, core, sub, zbuf)


def _sc_conv(A, B, dstf, srcf, sums, a_v, b_v, idxr, idxs, zbuf, acc,
             sem, sem2):
    core = lax.axis_index("c")
    sub = lax.axis_index("s")
    tile_off = (core * NS + sub) * EPT

    _zero_rows(zbuf, 128, D)
    _zero_acc(acc, sub, zbuf)
    plsc.subcore_barrier()

    def chunk(c, _):
        base = tile_off + c * EC
        pltpu.sync_copy(dstf.at[pl.ds(base, EC)], idxr)
        pltpu.sync_copy(srcf.at[pl.ds(base, EC)], idxs)
        g1 = pltpu.async_copy(A.at[idxr], a_v, sem)
        g2 = pltpu.async_copy(B.at[idxs], b_v, sem2)
        g1.wait()
        g2.wait()

        def edge(r, _):
            for j in range(D // VL):
                ds = pl.ds(j * VL, VL)
                z = a_v[r, ds] + b_v[r, ds]
                a_v[r, ds] = jnp.maximum(z, z * jnp.float32(0.01))
            return 0

        lax.fori_loop(0, EC, edge, 0)
        pltpu.sync_copy(a_v, acc.at[idxr], add=True)
        return 0

    lax.fori_loop(0, NCHUNK, chunk, 0)
    plsc.subcore_barrier()
    _copy_acc_out(acc, sums, core, sub, zbuf)


def _sc_final(S, Dd, dstf, srcf, eout, sq,
              s1, s2, d1, d2, idxr, idxs, sqb,
              sem, sem2, sem3, sem4):
    core = lax.axis_index("c")
    sub = lax.axis_index("s")
    tile_off = (core * NS + sub) * EPT

    def chunk(c, acc):
        base = tile_off + c * EC
        pltpu.sync_copy(dstf.at[pl.ds(base, EC)], idxr)
        pltpu.sync_copy(srcf.at[pl.ds(base, EC)], idxs)
        g1 = pltpu.async_copy(S.at[idxs], s1, sem)
        g2 = pltpu.async_copy(S.at[idxr], s2, sem2)
        g3 = pltpu.async_copy(Dd.at[idxs], d1, sem3)
        g4 = pltpu.async_copy(Dd.at[idxr], d2, sem4)
        g1.wait()
        g2.wait()
        g3.wait()
        g4.wait()

        def edge(r, a):
            for j in range(D // VL):
                ds = pl.ds(j * VL, VL)
                s1[r, ds] = s1[r, ds] + s2[r, ds]
                dv = d1[r, ds] - d2[r, ds]
                a = a + dv * dv
            return a

        acc = lax.fori_loop(0, EC, edge, acc)
        pltpu.sync_copy(s1, eout.at[pl.ds(base, EC)])
        return acc

    acc = lax.fori_loop(0, NCHUNK, chunk, jnp.zeros((VL,), jnp.float32))
    sqb[...] = acc
    wid = core * NS + sub
    pltpu.sync_copy(sqb, sq.at[pl.ds(wid * VL, VL)])


_count_kernel = pl.kernel(
    _sc_count,
    out_type=jax.ShapeDtypeStruct((NC * NNP, CW), jnp.float32),
    mesh=_mesh,
    scratch_types=[
        pltpu.VMEM((EC, CW), jnp.float32),
        pltpu.VMEM((EC,), jnp.int32),
        pltpu.VMEM((128, CW), jnp.float32),
        pltpu.VMEM_SHARED((NNP, CW), jnp.float32),
    ],
)

_init_kernel = pl.kernel(
    _sc_init,
    out_type=jax.ShapeDtypeStruct((NC * NNP, D), jnp.float32),
    mesh=_mesh,
    scratch_types=[
        pltpu.VMEM((EC, D), jnp.float32),
        pltpu.VMEM((EC,), jnp.int32),
        pltpu.VMEM((128, D), jnp.float32),
        pltpu.VMEM_SHARED((NNP, D), jnp.float32),
        pltpu.SemaphoreType.DMA,
    ],
)

_conv_kernel = pl.kernel(
    _sc_conv,
    out_type=jax.ShapeDtypeStruct((NC * NNP, D), jnp.float32),
    mesh=_mesh,
    scratch_types=[
        pltpu.VMEM((EC, D), jnp.float32),
        pltpu.VMEM((EC, D), jnp.float32),
        pltpu.VMEM((EC,), jnp.int32),
        pltpu.VMEM((EC,), jnp.int32),
        pltpu.VMEM((128, D), jnp.float32),
        pltpu.VMEM_SHARED((NNP, D), jnp.float32),
        pltpu.SemaphoreType.DMA,
        pltpu.SemaphoreType.DMA,
    ],
)

_final_kernel = pl.kernel(
    _sc_final,
    out_type=(
        jax.ShapeDtypeStruct((NE, D), jnp.float32),
        jax.ShapeDtypeStruct((NC * NS * VL,), jnp.float32),
    ),
    mesh=_mesh,
    scratch_types=[
        pltpu.VMEM((EC, D), jnp.float32),
        pltpu.VMEM((EC, D), jnp.float32),
        pltpu.VMEM((EC, D), jnp.float32),
        pltpu.VMEM((EC, D), jnp.float32),
        pltpu.VMEM((EC,), jnp.int32),
        pltpu.VMEM((EC,), jnp.int32),
        pltpu.VMEM((VL,), jnp.float32),
        pltpu.SemaphoreType.DMA,
        pltpu.SemaphoreType.DMA,
        pltpu.SemaphoreType.DMA,
        pltpu.SemaphoreType.DMA,
    ],
)


_BR = 1024
_GRID = NNP // _BR


def _tc_update_body(xp, sa, sb, ca, cb, wl, wr, bl, xn, p1, p2):
    c = ca[:, 0:1] + cb[:, 0:1]
    iv = 1.0 / jnp.maximum(c, 1.0)
    x = xp[...] + (sa[...] + sb[...]) * iv
    xn[...] = x
    p1[...] = jnp.dot(x, wl[...], preferred_element_type=jnp.float32) + bl[...]
    p2[...] = jnp.dot(x, wr[...], preferred_element_type=jnp.float32)


def _row_spec(w):
    return pl.BlockSpec((_BR, w), lambda i: (i, 0))


def _plane1_spec(w):
    return pl.BlockSpec((_BR, w), lambda i: (i + _GRID, 0))


def _full_spec(h, w):
    return pl.BlockSpec((h, w), lambda i: (0, 0))


_tc_update = pl.pallas_call(
    _tc_update_body,
    grid=(_GRID,),
    in_specs=[
        _row_spec(D), _row_spec(D), _plane1_spec(D),
        _row_spec(CW), _plane1_spec(CW),
        _full_spec(D, D), _full_spec(D, D), _full_spec(1, D),
    ],
    out_specs=[_row_spec(D), _row_spec(D), _row_spec(D)],
    out_shape=[
        jax.ShapeDtypeStruct((NNP, D), jnp.float32),
        jax.ShapeDtypeStruct((NNP, D), jnp.float32),
        jax.ShapeDtypeStruct((NNP, D), jnp.float32),
    ],
)


def kernel(edge_features, edge_index, W0, b0, W1, b1, W2, b2, We, be):
    src = edge_index[0]
    dst = edge_index[1]

    cnts = _count_kernel(dst)
    sums = _init_kernel(edge_features, dst)

    x = jnp.zeros((NNP, D), jnp.float32)
    for W, b in ((W0, b0), (W1, b1), (W2, b2)):
        x, a_t, b_t = _tc_update(x, sums, sums, cnts, cnts, W[:D], W[D:],
                                 b.reshape(1, D))
        sums = _conv_kernel(a_t, b_t, dst, src)

    w_sum = 0.5 * (We[:D] + We[D:])
    w_diff = We[:D] - We[D:]
    _, s_t, d_t = _tc_update(x, sums, sums, cnts, cnts, w_sum, w_diff,
                             (0.5 * be).reshape(1, D))
    edge_out, sq = _final_kernel(s_t, d_t, dst, src)
    side_loss = jnp.sum(sq) / jnp.float32(NE * D)
    return edge_out, side_loss
